# Initial kernel scaffold; baseline (speedup 1.0000x reference)
import jax
import jax.numpy as jnp
from jax.experimental import pallas as pl


def kernel(src, tgt):
    raise NotImplementedError("write your pallas kernel here")


# speedup vs baseline: 1.3818x; 1.3818x over previous
import functools

import jax
import jax.numpy as jnp
from jax import lax
from jax.experimental import pallas as pl
from jax.experimental.pallas import tpu as pltpu
from jax.experimental.pallas import tpu_sc as plsc

_B = 1024
_V = 100000
_K = 20
_N = _B * _V

_NC = 2
_NS = 16
_NW = _NC * _NS
_L = 16

_PER_W = (_B * _K) // _NW
_NCH = _PER_W // _L
_IDX_ROWS = _PER_W // 128

_GW = 128
_GROWS = _N // _GW

_RV = 25000
_CV = 4096
_BR = 200
_GRID = _RV // _BR


def _sc_gather_body(src128, tgtf, roff, lrid, out,
                    tgt_v, roff_v, lrid_v, idx_v, lane_v, rows_v, val_v, sem):
    c = lax.axis_index("c")
    s = lax.axis_index("s")
    wid = s * _NC + c
    base = wid * _PER_W
    pltpu.sync_copy(tgtf.at[pl.ds(base, _PER_W)], tgt_v)
    pltpu.sync_copy(roff.at[pl.ds(base, _PER_W)], roff_v)
    pltpu.sync_copy(lrid, lrid_v)
    for ch in range(_NCH):
        sl = pl.ds(ch * _L, _L)
        flat = tgt_v[sl] + roff_v[sl]
        r128 = lax.shift_right_logical(flat, 7)
        lane = lax.bitwise_and(flat, _GW - 1)
        idx_v[ch // 8, pl.ds((ch % 8) * _L, _L)] = r128
        lane_v[sl] = lane
    copies = [
        pltpu.async_copy(src128.at[idx_v.at[q]],
                         rows_v.at[pl.ds(q * 128, 128)], sem)
        for q in range(_IDX_ROWS)
    ]
    for cp in copies:
        cp.wait()
    for ch in range(_NCH):
        sl = pl.ds(ch * _L, _L)
        val_v[sl] = plsc.load_gather(rows_v, [lrid_v[sl], lane_v[sl]])
    pltpu.sync_copy(val_v, out.at[pl.ds(base, _PER_W)])


@functools.cache
def _sc_gather():
    return pl.kernel(
        _sc_gather_body,
        out_type=jax.ShapeDtypeStruct((_B * _K,), jnp.float32),
        compiler_params=pltpu.CompilerParams(needs_layout_passes=False),
        mesh=plsc.VectorSubcoreMesh(
            core_axis_name="c", subcore_axis_name="s",
            num_cores=_NC, num_subcores=_NS),
        scratch_types=[
            pltpu.VMEM((_PER_W,), jnp.int32),
            pltpu.VMEM((_PER_W,), jnp.int32),
            pltpu.VMEM((_PER_W,), jnp.int32),
            pltpu.VMEM((_IDX_ROWS, 128), jnp.int32),
            pltpu.VMEM((_PER_W,), jnp.int32),
            pltpu.VMEM((_PER_W, _GW), jnp.float32),
            pltpu.VMEM((_PER_W,), jnp.float32),
            pltpu.SemaphoreType.DMA,
        ],
    )


def _dense_body(src_ref, tgt_ref, vals_ref, out_ref):
    step = pl.program_id(0)
    x = jnp.clip(src_ref[...], 1e-8, 1.0 - 1e-8)
    bsum = jnp.sum(x + jnp.log1p(jnp.exp(-x)))

    @pl.when(step == 0)
    def _init():
        out_ref[...] = jnp.zeros_like(out_ref)

    out_ref[...] += bsum

    @pl.when(step == _GRID - 1)
    def _fin():
        t = tgt_ref[...]
        v = jnp.clip(vals_ref[...], 1e-8, 1.0 - 1e-8)
        cols = [t[:, k] for k in range(_K)]
        vcols = [v[:, k] for k in range(_K)]
        corr = jnp.sum(vcols[0])
        for k in range(1, _K):
            keep = cols[0] != cols[k]
            for j in range(1, k):
                keep &= cols[j] != cols[k]
            corr += jnp.sum(jnp.where(keep, vcols[k], 0.0))
        out_ref[...] = (out_ref[...] - corr) * (1.0 / _N)


def _dense_call(srcv, tgt, vals):
    return pl.pallas_call(
        _dense_body,
        grid=(_GRID,),
        in_specs=[
            pl.BlockSpec((_BR, _CV), lambda i: (i, 0)),
            pl.BlockSpec((_B, _K), lambda i: (0, 0)),
            pl.BlockSpec((_B, _K), lambda i: (0, 0)),
        ],
        out_specs=pl.BlockSpec((1, 1), lambda i: (0, 0)),
        out_shape=jax.ShapeDtypeStruct((1, 1), jnp.float32),
    )(srcv, tgt, vals)


def kernel(src, tgt):
    src128 = src.reshape(_GROWS, _GW)
    tgtf = tgt.reshape(-1).astype(jnp.int32)
    roff = ((jnp.arange(_B * _K, dtype=jnp.int32) // _K) * _V).astype(jnp.int32)
    lrid = jnp.arange(_PER_W, dtype=jnp.int32)
    vals = _sc_gather()(src128, tgtf, roff, lrid).reshape(_B, _K)
    srcv = src.reshape(_RV, _CV)
    out = _dense_call(srcv, tgt.astype(jnp.int32), vals)
    return out[0, 0]


# speedup vs baseline: 2.1871x; 1.5827x over previous
import functools

import jax
import jax.numpy as jnp
from jax import lax
from jax.experimental import pallas as pl
from jax.experimental.pallas import tpu as pltpu
from jax.experimental.pallas import tpu_sc as plsc

_B = 1024
_V = 100000
_K = 20
_N = _B * _V

_NC = 2
_NS = 16
_NW = _NC * _NS
_L = 16

_PER_W = (_B * _K) // _NW
_NCH = _PER_W // _L
_IDX_ROWS = _PER_W // 128

_GW = 128
_GROWS = _N // _GW

_CW = 2048
_GRID = (_V + _CW - 1) // _CW

_PC = (0.6931471596930971, 0.5000011560316415, 0.12498464848034356,
       8.310228184892147e-05, -0.005426855422417802,
       0.00028751330110348837, 0.00018498514140021503)


def _sc_gather_body(src128, tgtf, roff, lrid, out,
                    tgt_v, roff_v, lrid_v, idx_v, lane_v, rows_v, val_v, sem):
    c = lax.axis_index("c")
    s = lax.axis_index("s")
    wid = s * _NC + c
    base = wid * _PER_W
    pltpu.sync_copy(tgtf.at[pl.ds(base, _PER_W)], tgt_v)
    pltpu.sync_copy(roff.at[pl.ds(base, _PER_W)], roff_v)
    pltpu.sync_copy(lrid, lrid_v)
    for ch in range(_NCH):
        sl = pl.ds(ch * _L, _L)
        flat = tgt_v[sl] + roff_v[sl]
        r128 = lax.shift_right_logical(flat, 7)
        lane = lax.bitwise_and(flat, _GW - 1)
        idx_v[ch // 8, pl.ds((ch % 8) * _L, _L)] = r128
        lane_v[sl] = lane
    copies = [
        pltpu.async_copy(src128.at[idx_v.at[q]],
                         rows_v.at[pl.ds(q * 128, 128)], sem)
        for q in range(_IDX_ROWS)
    ]
    for cp in copies:
        cp.wait()
    for ch in range(_NCH):
        sl = pl.ds(ch * _L, _L)
        val_v[sl] = plsc.load_gather(rows_v, [lrid_v[sl], lane_v[sl]])
    pltpu.sync_copy(val_v, out.at[pl.ds(base, _PER_W)])


@functools.cache
def _sc_gather():
    return pl.kernel(
        _sc_gather_body,
        out_type=jax.ShapeDtypeStruct((_B * _K,), jnp.float32),
        compiler_params=pltpu.CompilerParams(needs_layout_passes=False),
        mesh=plsc.VectorSubcoreMesh(
            core_axis_name="c", subcore_axis_name="s",
            num_cores=_NC, num_subcores=_NS),
        scratch_types=[
            pltpu.VMEM((_PER_W,), jnp.int32),
            pltpu.VMEM((_PER_W,), jnp.int32),
            pltpu.VMEM((_PER_W,), jnp.int32),
            pltpu.VMEM((_IDX_ROWS, 128), jnp.int32),
            pltpu.VMEM((_PER_W,), jnp.int32),
            pltpu.VMEM((_PER_W, _GW), jnp.float32),
            pltpu.VMEM((_PER_W,), jnp.float32),
            pltpu.SemaphoreType.DMA,
        ],
    )


def _dense_body(src_ref, tgt_ref, vals_ref, out_ref):
    step = pl.program_id(0)
    x = jnp.clip(src_ref[...], 1e-8, 1.0 - 1e-8)
    g = jnp.float32(_PC[6])
    for c in range(5, -1, -1):
        g = g * x + jnp.float32(_PC[c])
    col = step * _CW + lax.broadcasted_iota(jnp.int32, x.shape, 1)
    bsum = jnp.sum(jnp.where(col < _V, g, 0.0))

    @pl.when(step == 0)
    def _init():
        out_ref[...] = jnp.zeros_like(out_ref)

    out_ref[...] += bsum

    @pl.when(step == _GRID - 1)
    def _fin():
        t = tgt_ref[...]
        v = jnp.clip(vals_ref[...], 1e-8, 1.0 - 1e-8)
        cols = [t[:, k] for k in range(_K)]
        vcols = [v[:, k] for k in range(_K)]
        corr = jnp.sum(vcols[0])
        for k in range(1, _K):
            keep = cols[0] != cols[k]
            for j in range(1, k):
                keep &= cols[j] != cols[k]
            corr += jnp.sum(jnp.where(keep, vcols[k], 0.0))
        out_ref[...] = (out_ref[...] - corr) * (1.0 / _N)


def _dense_call(src, tgt, vals):
    return pl.pallas_call(
        _dense_body,
        grid=(_GRID,),
        in_specs=[
            pl.BlockSpec((_B, _CW), lambda i: (0, i)),
            pl.BlockSpec((_B, _K), lambda i: (0, 0)),
            pl.BlockSpec((_B, _K), lambda i: (0, 0)),
        ],
        out_specs=pl.BlockSpec((1, 1), lambda i: (0, 0)),
        out_shape=jax.ShapeDtypeStruct((1, 1), jnp.float32),
    )(src, tgt, vals)


def kernel(src, tgt):
    src128 = src.reshape(_GROWS, _GW)
    tgtf = tgt.reshape(-1).astype(jnp.int32)
    roff = ((jnp.arange(_B * _K, dtype=jnp.int32) // _K) * _V).astype(jnp.int32)
    lrid = jnp.arange(_PER_W, dtype=jnp.int32)
    vals = _sc_gather()(src128, tgtf, roff, lrid).reshape(_B, _K)
    out = _dense_call(src, tgt.astype(jnp.int32), vals)
    return out[0, 0]
